# Initial kernel scaffold; baseline (speedup 1.0000x reference)
import jax
import jax.numpy as jnp
from jax.experimental import pallas as pl


def kernel(fatoms, fbonds, agraph, bgraph, W_i, W_h, W_o, b_o):
    raise NotImplementedError("write your pallas kernel here")


# speedup vs baseline: 2.8130x; 2.8130x over previous
import functools

import jax
import jax.numpy as jnp
from jax import lax
from jax.experimental import pallas as pl
from jax.experimental.pallas import tpu as pltpu
from jax.experimental.pallas import tpu_sc as plsc

_NC = 2
_NS = 16
_NW = _NC * _NS
_LANES = 16
_CHUNK = 128


def _mm_in(fbonds_p, W_i, block_rows=2048, interpret=False):
    Ep, K = fbonds_p.shape
    H = W_i.shape[0]
    grid = Ep // block_rows

    def body(x_ref, w_ref, o_ref):
        o_ref[...] = lax.dot_general(
            x_ref[...], w_ref[...], (((1,), (1,)), ((), ())),
            preferred_element_type=jnp.float32)

    return pl.pallas_call(
        body,
        grid=(grid,),
        in_specs=[
            pl.BlockSpec((block_rows, K), lambda i: (i, 0)),
            pl.BlockSpec((H, K), lambda i: (0, 0)),
        ],
        out_specs=pl.BlockSpec((block_rows, H), lambda i: (i, 0)),
        out_shape=jax.ShapeDtypeStruct((Ep, H), jnp.float32),
        interpret=interpret,
    )(fbonds_p, W_i)


def _mm_h(binput, gs, W_h, block_rows=2048, interpret=False):
    Ep, H = binput.shape
    grid = Ep // block_rows

    def body(b_ref, g_ref, w_ref, o_ref):
        o_ref[...] = b_ref[...] + lax.dot_general(
            g_ref[...], w_ref[...], (((1,), (1,)), ((), ())),
            preferred_element_type=jnp.float32)

    return pl.pallas_call(
        body,
        grid=(grid,),
        in_specs=[
            pl.BlockSpec((block_rows, H), lambda i: (i, 0)),
            pl.BlockSpec((block_rows, H), lambda i: (i, 0)),
            pl.BlockSpec((H, H), lambda i: (0, 0)),
        ],
        out_specs=pl.BlockSpec((block_rows, H), lambda i: (i, 0)),
        out_shape=jax.ShapeDtypeStruct((Ep, H), jnp.float32),
        interpret=interpret,
    )(binput, gs, W_h)


def _mm_out(fatoms, ga, WoA, WoH, b_o2, block_rows=2000, interpret=False):
    N, AF = fatoms.shape
    H = WoH.shape[0]
    grid = N // block_rows

    def body(fa_ref, g_ref, wa_ref, wh_ref, b_ref, o_ref):
        i = pl.program_id(0)
        h = lax.dot_general(fa_ref[...], wa_ref[...], (((1,), (1,)), ((), ())),
                            preferred_element_type=jnp.float32)
        h = h + lax.dot_general(g_ref[...], wh_ref[...], (((1,), (1,)), ((), ())),
                                preferred_element_type=jnp.float32)
        h = jnp.maximum(h + b_ref[...], 0.0)
        part = jnp.sum(h, axis=0, keepdims=True)

        @pl.when(i == 0)
        def _zero():
            o_ref[...] = jnp.zeros_like(o_ref)

        o_ref[...] += part

        @pl.when(i == grid - 1)
        def _scale():
            o_ref[...] = o_ref[...] * (1.0 / N)

    return pl.pallas_call(
        body,
        grid=(grid,),
        in_specs=[
            pl.BlockSpec((block_rows, AF), lambda i: (i, 0)),
            pl.BlockSpec((block_rows, H), lambda i: (i, 0)),
            pl.BlockSpec(WoA.shape, lambda i: (0, 0)),
            pl.BlockSpec(WoH.shape, lambda i: (0, 0)),
            pl.BlockSpec((1, H), lambda i: (0, 0)),
        ],
        out_specs=pl.BlockSpec((1, H), lambda i: (0, 0)),
        out_shape=jax.ShapeDtypeStruct((1, H), jnp.float32),
        interpret=interpret,
    )(fatoms, ga, WoA, WoH, b_o2)


def _gather_relu_sum(msg_p, nbrT_p):
    Ep, H = msg_p.shape
    NB, R = nbrT_p.shape
    rows_per_worker = R // _NW
    n_chunks = rows_per_worker // _CHUNK
    ngroups = H // _LANES

    mesh = plsc.VectorSubcoreMesh(core_axis_name="c", subcore_axis_name="s")

    @functools.partial(
        pl.kernel,
        out_type=jax.ShapeDtypeStruct((R, H), jnp.float32),
        mesh=mesh,
        scratch_types=[
            pltpu.VMEM((NB, _CHUNK), jnp.int32),
            pltpu.VMEM((NB, _CHUNK, H), jnp.float32),
            pltpu.VMEM((_CHUNK, H), jnp.float32),
            pltpu.SemaphoreType.DMA,
        ],
    )
    def k(msg_hbm, nbr_hbm, out_hbm, idx_v, buf_v, acc_v, sem):
        wid = lax.axis_index("s") * _NC + lax.axis_index("c")
        base = wid * rows_per_worker

        def chunk_body(t, carry):
            r0 = base + t * _CHUNK
            pltpu.sync_copy(nbr_hbm.at[:, pl.ds(r0, _CHUNK)], idx_v)
            copies = [
                pltpu.async_copy(msg_hbm.at[idx_v.at[kk]], buf_v.at[kk], sem)
                for kk in range(NB)
            ]
            for cp in copies:
                cp.wait()

            def row_body(i, c2):
                for g in range(ngroups):
                    sl = pl.ds(g * _LANES, _LANES)
                    s = jnp.maximum(buf_v[0, i, sl], 0.0)
                    for kk in range(1, NB):
                        s = s + jnp.maximum(buf_v[kk, i, sl], 0.0)
                    acc_v[i, sl] = s
                return c2

            lax.fori_loop(0, _CHUNK, row_body, 0)
            pltpu.sync_copy(acc_v, out_hbm.at[pl.ds(r0, _CHUNK), :])
            return carry

        lax.fori_loop(0, n_chunks, chunk_body, 0)

    return k(msg_p, nbrT_p)


def kernel(fatoms, fbonds, agraph, bgraph, W_i, W_h, W_o, b_o):
    N, AF = fatoms.shape
    E, K = fbonds.shape
    H = W_i.shape[0]
    NB = bgraph.shape[1]
    tile = _NW * _CHUNK
    Ep = -(-E // tile) * tile
    Np = -(-N // tile) * tile

    bgT = jnp.zeros((NB, Ep), jnp.int32).at[:, :E].set(
        bgraph.astype(jnp.int32).T)
    agT = jnp.zeros((NB, Np), jnp.int32).at[:, :N].set(
        agraph.astype(jnp.int32).T)
    fbonds_p = jnp.zeros((Ep, K), jnp.float32).at[:E].set(fbonds)

    binput = _mm_in(fbonds_p, W_i)
    pre = binput
    for _ in range(2):
        gs = _gather_relu_sum(pre, bgT)
        pre = _mm_h(binput, gs, W_h)
    ga = _gather_relu_sum(pre, agT)[:N]
    return _mm_out(fatoms, ga, W_o[:, :AF], W_o[:, AF:], b_o.reshape(1, H))


# speedup vs baseline: 3.2467x; 1.1542x over previous
import functools

import jax
import jax.numpy as jnp
from jax import lax
from jax.experimental import pallas as pl
from jax.experimental.pallas import tpu as pltpu
from jax.experimental.pallas import tpu_sc as plsc

_NC = 2
_NS = 16
_NW = _NC * _NS
_LANES = 16
_CHUNK = 128


def _mm_in(fbonds, W_i, block_rows=2000, interpret=False):
    E, K = fbonds.shape
    H = W_i.shape[0]
    grid = E // block_rows

    def body(x_ref, w_ref, o_ref, m_ref):
        b = lax.dot_general(
            x_ref[...], w_ref[...], (((1,), (1,)), ((), ())),
            preferred_element_type=jnp.float32)
        o_ref[...] = b
        m_ref[...] = jnp.maximum(b, 0.0)

    return pl.pallas_call(
        body,
        grid=(grid,),
        in_specs=[
            pl.BlockSpec((block_rows, K), lambda i: (i, 0)),
            pl.BlockSpec((H, K), lambda i: (0, 0)),
        ],
        out_specs=[
            pl.BlockSpec((block_rows, H), lambda i: (i, 0)),
            pl.BlockSpec((block_rows, H), lambda i: (i, 0)),
        ],
        out_shape=[
            jax.ShapeDtypeStruct((E, H), jnp.float32),
            jax.ShapeDtypeStruct((E, H), jnp.float32),
        ],
        interpret=interpret,
    )(fbonds, W_i)


def _mm_h(binput, gs_p, W_h, block_rows=2000, interpret=False):
    E, H = binput.shape
    grid = E // block_rows

    def body(b_ref, g_ref, w_ref, o_ref):
        o_ref[...] = jnp.maximum(
            b_ref[...] + lax.dot_general(
                g_ref[...], w_ref[...], (((1,), (1,)), ((), ())),
                preferred_element_type=jnp.float32),
            0.0)

    return pl.pallas_call(
        body,
        grid=(grid,),
        in_specs=[
            pl.BlockSpec((block_rows, H), lambda i: (i, 0)),
            pl.BlockSpec((block_rows, H), lambda i: (i, 0)),
            pl.BlockSpec((H, H), lambda i: (0, 0)),
        ],
        out_specs=pl.BlockSpec((block_rows, H), lambda i: (i, 0)),
        out_shape=jax.ShapeDtypeStruct((E, H), jnp.float32),
        interpret=interpret,
    )(binput, gs_p, W_h)


def _mm_out(fatoms, ga_p, WoA, WoH, b_o2, block_rows=2000, interpret=False):
    N, AF = fatoms.shape
    H = WoH.shape[0]
    grid = N // block_rows

    def body(fa_ref, g_ref, wa_ref, wh_ref, b_ref, o_ref):
        i = pl.program_id(0)
        h = lax.dot_general(fa_ref[...], wa_ref[...], (((1,), (1,)), ((), ())),
                            preferred_element_type=jnp.float32)
        h = h + lax.dot_general(g_ref[...], wh_ref[...], (((1,), (1,)), ((), ())),
                                preferred_element_type=jnp.float32)
        h = jnp.maximum(h + b_ref[...], 0.0)
        part = jnp.sum(h, axis=0, keepdims=True)

        @pl.when(i == 0)
        def _zero():
            o_ref[...] = jnp.zeros_like(o_ref)

        o_ref[...] += part

        @pl.when(i == grid - 1)
        def _scale():
            o_ref[...] = o_ref[...] * (1.0 / N)

    return pl.pallas_call(
        body,
        grid=(grid,),
        in_specs=[
            pl.BlockSpec((block_rows, AF), lambda i: (i, 0)),
            pl.BlockSpec((block_rows, H), lambda i: (i, 0)),
            pl.BlockSpec(WoA.shape, lambda i: (0, 0)),
            pl.BlockSpec(WoH.shape, lambda i: (0, 0)),
            pl.BlockSpec((1, H), lambda i: (0, 0)),
        ],
        out_specs=pl.BlockSpec((1, H), lambda i: (0, 0)),
        out_shape=jax.ShapeDtypeStruct((1, H), jnp.float32),
        interpret=interpret,
    )(fatoms, ga_p, WoA, WoH, b_o2)


def _gather_sum(msg, nbrT_p):
    E, H = msg.shape
    NB, Rp = nbrT_p.shape
    C = _CHUNK
    rpw = Rp // _NW
    n_chunks = rpw // C

    mesh = plsc.VectorSubcoreMesh(core_axis_name="c", subcore_axis_name="s")

    @functools.partial(
        pl.kernel,
        out_type=jax.ShapeDtypeStruct((Rp, H), jnp.float32),
        mesh=mesh,
        scratch_types=[
            pltpu.VMEM((NB, C), jnp.int32),
            pltpu.VMEM((C, H), jnp.float32),
            pltpu.SemaphoreType.DMA,
        ],
    )
    def k(msg_hbm, nbr_hbm, out_hbm, idxt_v, buf_v, sem):
        wid = lax.axis_index("s") * _NC + lax.axis_index("c")
        base = wid * rpw

        def chunk_body(t, carry):
            r0 = base + C * t
            pltpu.sync_copy(nbr_hbm.at[:, pl.ds(r0, C)], idxt_v)
            cp0 = pltpu.async_copy(msg_hbm.at[idxt_v.at[0]], buf_v, sem)
            cp0.wait()
            adds = [
                pltpu.async_copy(msg_hbm.at[idxt_v.at[kk]], buf_v, sem,
                                 add=True)
                for kk in range(1, NB)
            ]
            for cp in adds:
                cp.wait()
            pltpu.sync_copy(buf_v, out_hbm.at[pl.ds(r0, C), :])
            return carry

        lax.fori_loop(0, n_chunks, chunk_body, 0)

    return k(msg, nbrT_p)


def kernel(fatoms, fbonds, agraph, bgraph, W_i, W_h, W_o, b_o):
    N, AF = fatoms.shape
    E, K = fbonds.shape
    H = W_i.shape[0]
    NB = bgraph.shape[1]
    tile = _NW * _CHUNK
    Ep = -(-E // tile) * tile
    Np = -(-N // tile) * tile

    bT = jnp.zeros((NB, Ep), jnp.int32).at[:, :E].set(
        bgraph.astype(jnp.int32).T)
    aT = jnp.zeros((NB, Np), jnp.int32).at[:, :N].set(
        agraph.astype(jnp.int32).T)

    binput, msg = _mm_in(fbonds, W_i)
    for _ in range(2):
        gs_p = _gather_sum(msg, bT)
        msg = _mm_h(binput, gs_p, W_h)
    ga_p = _gather_sum(msg, aT)
    return _mm_out(fatoms, ga_p, W_o[:, :AF], W_o[:, AF:], b_o.reshape(1, H))


# speedup vs baseline: 3.2903x; 1.0134x over previous
import functools

import jax
import jax.numpy as jnp
from jax import lax
from jax.experimental import pallas as pl
from jax.experimental.pallas import tpu as pltpu
from jax.experimental.pallas import tpu_sc as plsc

_NC = 2
_NS = 16
_NW = _NC * _NS
_LANES = 16
_CHUNK = 128


def _mm_in(fbonds, W_i, block_rows=2000, interpret=False):
    E, K = fbonds.shape
    H = W_i.shape[0]
    grid = E // block_rows

    def body(x_ref, w_ref, o_ref, m_ref):
        b = lax.dot_general(
            x_ref[...], w_ref[...], (((1,), (1,)), ((), ())),
            preferred_element_type=jnp.float32)
        o_ref[...] = b
        m_ref[...] = jnp.maximum(b, 0.0)

    return pl.pallas_call(
        body,
        grid=(grid,),
        in_specs=[
            pl.BlockSpec((block_rows, K), lambda i: (i, 0)),
            pl.BlockSpec((H, K), lambda i: (0, 0)),
        ],
        out_specs=[
            pl.BlockSpec((block_rows, H), lambda i: (i, 0)),
            pl.BlockSpec((block_rows, H), lambda i: (i, 0)),
        ],
        out_shape=[
            jax.ShapeDtypeStruct((E, H), jnp.float32),
            jax.ShapeDtypeStruct((E, H), jnp.float32),
        ],
        interpret=interpret,
    )(fbonds, W_i)


def _mm_h(binput, gs_p, W_h, block_rows=2000, interpret=False):
    E, H = binput.shape
    grid = E // block_rows

    def body(b_ref, g_ref, w_ref, o_ref):
        o_ref[...] = jnp.maximum(
            b_ref[...] + lax.dot_general(
                g_ref[...], w_ref[...], (((1,), (1,)), ((), ())),
                preferred_element_type=jnp.float32),
            0.0)

    return pl.pallas_call(
        body,
        grid=(grid,),
        in_specs=[
            pl.BlockSpec((block_rows, H), lambda i: (i, 0)),
            pl.BlockSpec((block_rows, H), lambda i: (i, 0)),
            pl.BlockSpec((H, H), lambda i: (0, 0)),
        ],
        out_specs=pl.BlockSpec((block_rows, H), lambda i: (i, 0)),
        out_shape=jax.ShapeDtypeStruct((E, H), jnp.float32),
        interpret=interpret,
    )(binput, gs_p, W_h)


def _mm_out(fatoms, ga_p, WoA, WoH, b_o2, block_rows=2000, interpret=False):
    N, AF = fatoms.shape
    H = WoH.shape[0]
    grid = N // block_rows

    def body(fa_ref, g_ref, wa_ref, wh_ref, b_ref, o_ref):
        i = pl.program_id(0)
        h = lax.dot_general(fa_ref[...], wa_ref[...], (((1,), (1,)), ((), ())),
                            preferred_element_type=jnp.float32)
        h = h + lax.dot_general(g_ref[...], wh_ref[...], (((1,), (1,)), ((), ())),
                                preferred_element_type=jnp.float32)
        h = jnp.maximum(h + b_ref[...], 0.0)
        part = jnp.sum(h, axis=0, keepdims=True)

        @pl.when(i == 0)
        def _zero():
            o_ref[...] = jnp.zeros_like(o_ref)

        o_ref[...] += part

        @pl.when(i == grid - 1)
        def _scale():
            o_ref[...] = o_ref[...] * (1.0 / N)

    return pl.pallas_call(
        body,
        grid=(grid,),
        in_specs=[
            pl.BlockSpec((block_rows, AF), lambda i: (i, 0)),
            pl.BlockSpec((block_rows, H), lambda i: (i, 0)),
            pl.BlockSpec(WoA.shape, lambda i: (0, 0)),
            pl.BlockSpec(WoH.shape, lambda i: (0, 0)),
            pl.BlockSpec((1, H), lambda i: (0, 0)),
        ],
        out_specs=pl.BlockSpec((1, H), lambda i: (0, 0)),
        out_shape=jax.ShapeDtypeStruct((1, H), jnp.float32),
        interpret=interpret,
    )(fatoms, ga_p, WoA, WoH, b_o2)


def _gather_sum(msg, nbrT_p):
    E, H = msg.shape
    NB, Rp = nbrT_p.shape
    C = _CHUNK
    rpw = Rp // _NW
    n_chunks = rpw // C

    mesh = plsc.VectorSubcoreMesh(core_axis_name="c", subcore_axis_name="s")

    @functools.partial(
        pl.kernel,
        out_type=jax.ShapeDtypeStruct((Rp, H), jnp.float32),
        mesh=mesh,
        scratch_types=[
            pltpu.VMEM((2, NB, C), jnp.int32),
            pltpu.VMEM((2, C, H), jnp.float32),
            [pltpu.SemaphoreType.DMA] * 8,
        ],
    )
    def k(msg_hbm, nbr_hbm, out_hbm, idxt_v, buf_v, sems):
        isem = sems[0:2]
        g0sem = sems[2:4]
        gsem = sems[4:6]
        wsem = sems[6:8]
        wid = lax.axis_index("s") * _NC + lax.axis_index("c")
        base = wid * rpw
        n = n_chunks

        def r0_of(t):
            return pl.multiple_of(base + C * jnp.minimum(t, n - 1), C)

        def wait_idx(sem, s):
            pltpu.make_async_copy(
                nbr_hbm.at[:, pl.ds(base, C)], idxt_v.at[s], sem).wait()

        def wait_gather(sem, s):
            pltpu.make_async_copy(
                out_hbm.at[pl.ds(base, C), :], buf_v.at[s], sem).wait()

        def wait_wb(sem, s):
            pltpu.make_async_copy(
                buf_v.at[s], out_hbm.at[pl.ds(base, C), :], sem).wait()

        def fire_g0(t, s):
            return pltpu.async_copy(
                msg_hbm.at[idxt_v.at[s, 0]], buf_v.at[s], g0sem[s])

        def fire_adds(s):
            for kk in range(1, NB):
                pltpu.async_copy(msg_hbm.at[idxt_v.at[s, kk]], buf_v.at[s],
                                 gsem[s], add=True)

        def half(tt, s, guard):
            o = 1 - s
            wait_idx(isem[o], o)
            if guard is None:
                wait_wb(wsem[o], o)
            else:
                @pl.when(guard)
                def _():
                    wait_wb(wsem[o], o)
            fire_g0(tt + 1, o)
            for _ in range(NB - 1):
                wait_gather(gsem[s], s)
            pltpu.async_copy(buf_v.at[s], out_hbm.at[pl.ds(r0_of(tt), C), :],
                             wsem[s])
            pltpu.async_copy(nbr_hbm.at[:, pl.ds(r0_of(tt + 2), C)],
                             idxt_v.at[s], isem[s])
            wait_gather(g0sem[o], o)
            fire_adds(o)

        pltpu.sync_copy(nbr_hbm.at[:, pl.ds(base, C)], idxt_v.at[0])
        fire_g0(0, 0).wait()
        fire_adds(0)
        pltpu.async_copy(nbr_hbm.at[:, pl.ds(r0_of(1), C)], idxt_v.at[1],
                         isem[1])

        def pair_body(t2, carry):
            half(2 * t2, 0, guard=t2 > 0)
            half(2 * t2 + 1, 1, guard=None)
            return carry

        lax.fori_loop(0, n // 2, pair_body, 0)
        if n % 2:
            half(n - 1, (n - 1) % 2, guard=None)

        s_last = (n - 1) % 2
        o_last = 1 - s_last
        for _ in range(NB - 1):
            wait_gather(gsem[o_last], o_last)
        wait_wb(wsem[s_last], s_last)
        wait_idx(isem[s_last], s_last)

    return k(msg, nbrT_p)


def kernel(fatoms, fbonds, agraph, bgraph, W_i, W_h, W_o, b_o):
    N, AF = fatoms.shape
    E, K = fbonds.shape
    H = W_i.shape[0]
    NB = bgraph.shape[1]
    tile = _NW * _CHUNK
    Ep = -(-E // tile) * tile
    Np = -(-N // tile) * tile

    bT = jnp.zeros((NB, Ep), jnp.int32).at[:, :E].set(
        bgraph.astype(jnp.int32).T)
    aT = jnp.zeros((NB, Np), jnp.int32).at[:, :N].set(
        agraph.astype(jnp.int32).T)

    binput, msg = _mm_in(fbonds, W_i)
    for _ in range(2):
        gs_p = _gather_sum(msg, bT)
        msg = _mm_h(binput, gs_p, W_h)
    ga_p = _gather_sum(msg, aT)
    return _mm_out(fatoms, ga_p, W_o[:, :AF], W_o[:, AF:], b_o.reshape(1, H))


# speedup vs baseline: 6.9089x; 2.0998x over previous
import functools

import jax
import jax.numpy as jnp
from jax import lax
from jax.experimental import pallas as pl
from jax.experimental.pallas import tpu as pltpu
from jax.experimental.pallas import tpu_sc as plsc

_NC = 2
_NS = 16
_NW = _NC * _NS
_LANES = 16
_CHUNK = 128


def _mm_in(fbonds, W_i, block_rows=2000, interpret=False):
    E, K = fbonds.shape
    H = W_i.shape[0]
    grid = E // block_rows

    def body(x_ref, w_ref, o_ref, m_ref):
        b = lax.dot_general(
            x_ref[...], w_ref[...], (((1,), (1,)), ((), ())),
            preferred_element_type=jnp.float32)
        o_ref[...] = b
        m_ref[...] = jnp.maximum(b, 0.0)

    return pl.pallas_call(
        body,
        grid=(grid,),
        in_specs=[
            pl.BlockSpec((block_rows, K), lambda i: (i, 0)),
            pl.BlockSpec((H, K), lambda i: (0, 0)),
        ],
        out_specs=[
            pl.BlockSpec((block_rows, H), lambda i: (i, 0)),
            pl.BlockSpec((block_rows, H), lambda i: (i, 0)),
        ],
        out_shape=[
            jax.ShapeDtypeStruct((E, H), jnp.float32),
            jax.ShapeDtypeStruct((E, H), jnp.float32),
        ],
        interpret=interpret,
    )(fbonds, W_i)


def _mm_h(binput, gs_p, W_h, block_rows=2000, interpret=False):
    E, H = binput.shape
    grid = E // block_rows

    def body(b_ref, g_ref, w_ref, o_ref):
        o_ref[...] = jnp.maximum(
            b_ref[...] + lax.dot_general(
                g_ref[...], w_ref[...], (((1,), (1,)), ((), ())),
                preferred_element_type=jnp.float32),
            0.0)

    return pl.pallas_call(
        body,
        grid=(grid,),
        in_specs=[
            pl.BlockSpec((block_rows, H), lambda i: (i, 0)),
            pl.BlockSpec((block_rows, H), lambda i: (i, 0)),
            pl.BlockSpec((H, H), lambda i: (0, 0)),
        ],
        out_specs=pl.BlockSpec((block_rows, H), lambda i: (i, 0)),
        out_shape=jax.ShapeDtypeStruct((E, H), jnp.float32),
        interpret=interpret,
    )(binput, gs_p, W_h)


def _mm_out(fatoms, ga_p, WoA, WoH, b_o2, block_rows=2000, interpret=False):
    N, AF = fatoms.shape
    H = WoH.shape[0]
    grid = N // block_rows

    def body(fa_ref, g_ref, wa_ref, wh_ref, b_ref, o_ref):
        i = pl.program_id(0)
        h = lax.dot_general(fa_ref[...], wa_ref[...], (((1,), (1,)), ((), ())),
                            preferred_element_type=jnp.float32)
        h = h + lax.dot_general(g_ref[...], wh_ref[...], (((1,), (1,)), ((), ())),
                                preferred_element_type=jnp.float32)
        h = jnp.maximum(h + b_ref[...], 0.0)
        part = jnp.sum(h, axis=0, keepdims=True)

        @pl.when(i == 0)
        def _zero():
            o_ref[...] = jnp.zeros_like(o_ref)

        o_ref[...] += part

        @pl.when(i == grid - 1)
        def _scale():
            o_ref[...] = o_ref[...] * (1.0 / N)

    return pl.pallas_call(
        body,
        grid=(grid,),
        in_specs=[
            pl.BlockSpec((block_rows, AF), lambda i: (i, 0)),
            pl.BlockSpec((block_rows, H), lambda i: (i, 0)),
            pl.BlockSpec(WoA.shape, lambda i: (0, 0)),
            pl.BlockSpec(WoH.shape, lambda i: (0, 0)),
            pl.BlockSpec((1, H), lambda i: (0, 0)),
        ],
        out_specs=pl.BlockSpec((1, H), lambda i: (0, 0)),
        out_shape=jax.ShapeDtypeStruct((1, H), jnp.float32),
        interpret=interpret,
    )(fatoms, ga_p, WoA, WoH, b_o2)


def _gather_sum(msg, nbrT_p):
    E, H = msg.shape
    NB, Rp = nbrT_p.shape
    C = _CHUNK
    rpw = Rp // _NW
    n_chunks = rpw // C

    mesh = plsc.VectorSubcoreMesh(core_axis_name="c", subcore_axis_name="s")

    @functools.partial(
        pl.kernel,
        out_type=jax.ShapeDtypeStruct((Rp, H), jnp.float32),
        mesh=mesh,
        scratch_types=[
            pltpu.VMEM((2, NB, C), jnp.int32),
            pltpu.VMEM((2, C, H), jnp.float32),
            [pltpu.SemaphoreType.DMA] * 8,
        ],
    )
    def k(msg_hbm, nbr_hbm, out_hbm, idxt_v, buf_v, sems):
        isem = sems[0:2]
        g0sem = sems[2:4]
        gsem = sems[4:6]
        wsem = sems[6:8]
        wid = lax.axis_index("s") * _NC + lax.axis_index("c")
        base = wid * rpw
        n = n_chunks

        def r0_of(t):
            return pl.multiple_of(base + C * jnp.minimum(t, n - 1), C)

        def wait_idx(sem, s):
            pltpu.make_async_copy(
                nbr_hbm.at[:, pl.ds(base, C)], idxt_v.at[s], sem).wait()

        def wait_gather(sem, s):
            pltpu.make_async_copy(
                out_hbm.at[pl.ds(base, C), :], buf_v.at[s], sem).wait()

        def wait_wb(sem, s):
            pltpu.make_async_copy(
                buf_v.at[s], out_hbm.at[pl.ds(base, C), :], sem).wait()

        def fire_g0(t, s):
            return pltpu.async_copy(
                msg_hbm.at[idxt_v.at[s, 0]], buf_v.at[s], g0sem[s])

        def fire_adds(s):
            for kk in range(1, NB):
                pltpu.async_copy(msg_hbm.at[idxt_v.at[s, kk]], buf_v.at[s],
                                 gsem[s], add=True)

        def half(tt, s, guard):
            o = 1 - s
            wait_idx(isem[o], o)
            if guard is None:
                wait_wb(wsem[o], o)
            else:
                @pl.when(guard)
                def _():
                    wait_wb(wsem[o], o)
            fire_g0(tt + 1, o)
            for _ in range(NB - 1):
                wait_gather(gsem[s], s)
            pltpu.async_copy(buf_v.at[s], out_hbm.at[pl.ds(r0_of(tt), C), :],
                             wsem[s])
            pltpu.async_copy(nbr_hbm.at[:, pl.ds(r0_of(tt + 2), C)],
                             idxt_v.at[s], isem[s])
            wait_gather(g0sem[o], o)
            fire_adds(o)

        pltpu.sync_copy(nbr_hbm.at[:, pl.ds(base, C)], idxt_v.at[0])
        fire_g0(0, 0).wait()
        fire_adds(0)
        pltpu.async_copy(nbr_hbm.at[:, pl.ds(r0_of(1), C)], idxt_v.at[1],
                         isem[1])

        def pair_body(t2, carry):
            half(2 * t2, 0, guard=t2 > 0)
            half(2 * t2 + 1, 1, guard=None)
            return carry

        lax.fori_loop(0, n // 2, pair_body, 0)
        if n % 2:
            half(n - 1, (n - 1) % 2, guard=None)

        s_last = (n - 1) % 2
        o_last = 1 - s_last
        for _ in range(NB - 1):
            wait_gather(gsem[o_last], o_last)
        wait_wb(wsem[s_last], s_last)
        wait_idx(isem[s_last], s_last)

    return k(msg, nbrT_p)


def kernel(fatoms, fbonds, agraph, bgraph, W_i, W_h, W_o, b_o):
    N, AF = fatoms.shape
    E, K = fbonds.shape
    H = W_i.shape[0]
    NB = bgraph.shape[1]
    tile = _NW * _CHUNK
    Ep = -(-E // tile) * tile
    Np = -(-N // tile) * tile

    padB = jnp.arange(NB * (Ep - E), dtype=jnp.int32).reshape(NB, Ep - E) % E
    padA = jnp.arange(NB * (Np - N), dtype=jnp.int32).reshape(NB, Np - N) % E
    bT = jnp.concatenate([bgraph.astype(jnp.int32).T, padB], axis=1)
    aT = jnp.concatenate([agraph.astype(jnp.int32).T, padA], axis=1)

    binput, msg = _mm_in(fbonds, W_i)
    for _ in range(2):
        gs_p = _gather_sum(msg, bT)
        msg = _mm_h(binput, gs_p, W_h)
    ga_p = _gather_sum(msg, aT)
    return _mm_out(fatoms, ga_p, W_o[:, :AF], W_o[:, AF:], b_o.reshape(1, H))


# speedup vs baseline: 6.9527x; 1.0063x over previous
import functools

import jax
import jax.numpy as jnp
from jax import lax
from jax.experimental import pallas as pl
from jax.experimental.pallas import tpu as pltpu
from jax.experimental.pallas import tpu_sc as plsc

_NC = 2
_NS = 16
_NW = _NC * _NS
_LANES = 16
_CHUNK = 128


def _mm_in(fbonds, W_i, block_rows=2000, interpret=False):
    E, K = fbonds.shape
    H = W_i.shape[0]
    grid = E // block_rows

    def body(x_ref, w_ref, m_ref):
        b = lax.dot_general(
            x_ref[...], w_ref[...], (((1,), (1,)), ((), ())),
            preferred_element_type=jnp.float32)
        m_ref[...] = jnp.maximum(b, 0.0)

    return pl.pallas_call(
        body,
        grid=(grid,),
        in_specs=[
            pl.BlockSpec((block_rows, K), lambda i: (i, 0)),
            pl.BlockSpec((H, K), lambda i: (0, 0)),
        ],
        out_specs=pl.BlockSpec((block_rows, H), lambda i: (i, 0)),
        out_shape=jax.ShapeDtypeStruct((E, H), jnp.float32),
        interpret=interpret,
    )(fbonds, W_i)


def _mm_h(fbonds, gs_p, W_i, W_h, block_rows=2000, interpret=False):
    E, K = fbonds.shape
    H = W_h.shape[0]
    grid = E // block_rows

    def body(x_ref, g_ref, wi_ref, wh_ref, o_ref):
        b = lax.dot_general(
            x_ref[...], wi_ref[...], (((1,), (1,)), ((), ())),
            preferred_element_type=jnp.float32)
        o_ref[...] = jnp.maximum(
            b + lax.dot_general(
                g_ref[...], wh_ref[...], (((1,), (1,)), ((), ())),
                preferred_element_type=jnp.float32),
            0.0)

    return pl.pallas_call(
        body,
        grid=(grid,),
        in_specs=[
            pl.BlockSpec((block_rows, K), lambda i: (i, 0)),
            pl.BlockSpec((block_rows, H), lambda i: (i, 0)),
            pl.BlockSpec((H, K), lambda i: (0, 0)),
            pl.BlockSpec((H, H), lambda i: (0, 0)),
        ],
        out_specs=pl.BlockSpec((block_rows, H), lambda i: (i, 0)),
        out_shape=jax.ShapeDtypeStruct((E, H), jnp.float32),
        interpret=interpret,
    )(fbonds, gs_p, W_i, W_h)


def _mm_out(fatoms, ga_p, WoA, WoH, b_o2, block_rows=2000, interpret=False):
    N, AF = fatoms.shape
    H = WoH.shape[0]
    grid = N // block_rows

    def body(fa_ref, g_ref, wa_ref, wh_ref, b_ref, o_ref):
        i = pl.program_id(0)
        h = lax.dot_general(fa_ref[...], wa_ref[...], (((1,), (1,)), ((), ())),
                            preferred_element_type=jnp.float32)
        h = h + lax.dot_general(g_ref[...], wh_ref[...], (((1,), (1,)), ((), ())),
                                preferred_element_type=jnp.float32)
        h = jnp.maximum(h + b_ref[...], 0.0)
        part = jnp.sum(h, axis=0, keepdims=True)

        @pl.when(i == 0)
        def _zero():
            o_ref[...] = jnp.zeros_like(o_ref)

        o_ref[...] += part

        @pl.when(i == grid - 1)
        def _scale():
            o_ref[...] = o_ref[...] * (1.0 / N)

    return pl.pallas_call(
        body,
        grid=(grid,),
        in_specs=[
            pl.BlockSpec((block_rows, AF), lambda i: (i, 0)),
            pl.BlockSpec((block_rows, H), lambda i: (i, 0)),
            pl.BlockSpec(WoA.shape, lambda i: (0, 0)),
            pl.BlockSpec(WoH.shape, lambda i: (0, 0)),
            pl.BlockSpec((1, H), lambda i: (0, 0)),
        ],
        out_specs=pl.BlockSpec((1, H), lambda i: (0, 0)),
        out_shape=jax.ShapeDtypeStruct((1, H), jnp.float32),
        interpret=interpret,
    )(fatoms, ga_p, WoA, WoH, b_o2)


def _gather_sum(msg, nbrT_p):
    E, H = msg.shape
    NB, Rp = nbrT_p.shape
    C = _CHUNK
    rpw = Rp // _NW
    n_chunks = rpw // C

    mesh = plsc.VectorSubcoreMesh(core_axis_name="c", subcore_axis_name="s")

    @functools.partial(
        pl.kernel,
        out_type=jax.ShapeDtypeStruct((Rp, H), jnp.float32),
        mesh=mesh,
        scratch_types=[
            pltpu.VMEM((2, NB, C), jnp.int32),
            pltpu.VMEM((2, C, H), jnp.float32),
            [pltpu.SemaphoreType.DMA] * 8,
        ],
    )
    def k(msg_hbm, nbr_hbm, out_hbm, idxt_v, buf_v, sems):
        isem = sems[0:2]
        g0sem = sems[2:4]
        gsem = sems[4:6]
        wsem = sems[6:8]
        wid = lax.axis_index("s") * _NC + lax.axis_index("c")
        base = wid * rpw
        n = n_chunks

        def r0_of(t):
            return pl.multiple_of(base + C * jnp.minimum(t, n - 1), C)

        def wait_idx(sem, s):
            pltpu.make_async_copy(
                nbr_hbm.at[:, pl.ds(base, C)], idxt_v.at[s], sem).wait()

        def wait_gather(sem, s):
            pltpu.make_async_copy(
                out_hbm.at[pl.ds(base, C), :], buf_v.at[s], sem).wait()

        def wait_wb(sem, s):
            pltpu.make_async_copy(
                buf_v.at[s], out_hbm.at[pl.ds(base, C), :], sem).wait()

        def fire_g0(t, s):
            return pltpu.async_copy(
                msg_hbm.at[idxt_v.at[s, 0]], buf_v.at[s], g0sem[s])

        def fire_adds(s):
            for kk in range(1, NB):
                pltpu.async_copy(msg_hbm.at[idxt_v.at[s, kk]], buf_v.at[s],
                                 gsem[s], add=True)

        def half(tt, s, guard):
            o = 1 - s
            wait_idx(isem[o], o)
            if guard is None:
                wait_wb(wsem[o], o)
            else:
                @pl.when(guard)
                def _():
                    wait_wb(wsem[o], o)
            fire_g0(tt + 1, o)
            for _ in range(NB - 1):
                wait_gather(gsem[s], s)
            pltpu.async_copy(buf_v.at[s], out_hbm.at[pl.ds(r0_of(tt), C), :],
                             wsem[s])
            pltpu.async_copy(nbr_hbm.at[:, pl.ds(r0_of(tt + 2), C)],
                             idxt_v.at[s], isem[s])
            wait_gather(g0sem[o], o)
            fire_adds(o)

        pltpu.sync_copy(nbr_hbm.at[:, pl.ds(base, C)], idxt_v.at[0])
        fire_g0(0, 0).wait()
        fire_adds(0)
        pltpu.async_copy(nbr_hbm.at[:, pl.ds(r0_of(1), C)], idxt_v.at[1],
                         isem[1])

        def pair_body(t2, carry):
            half(2 * t2, 0, guard=t2 > 0)
            half(2 * t2 + 1, 1, guard=None)
            return carry

        lax.fori_loop(0, n // 2, pair_body, 0)
        if n % 2:
            half(n - 1, (n - 1) % 2, guard=None)

        s_last = (n - 1) % 2
        o_last = 1 - s_last
        for _ in range(NB - 1):
            wait_gather(gsem[o_last], o_last)
        wait_wb(wsem[s_last], s_last)
        wait_idx(isem[s_last], s_last)

    return k(msg, nbrT_p)


def kernel(fatoms, fbonds, agraph, bgraph, W_i, W_h, W_o, b_o):
    N, AF = fatoms.shape
    E, K = fbonds.shape
    H = W_i.shape[0]
    NB = bgraph.shape[1]
    tile = _NW * _CHUNK
    Ep = -(-E // tile) * tile
    Np = -(-N // tile) * tile

    padB = jnp.arange(NB * (Ep - E), dtype=jnp.int32).reshape(NB, Ep - E) % E
    padA = jnp.arange(NB * (Np - N), dtype=jnp.int32).reshape(NB, Np - N) % E
    bT = jnp.concatenate([bgraph.astype(jnp.int32).T, padB], axis=1)
    aT = jnp.concatenate([agraph.astype(jnp.int32).T, padA], axis=1)

    msg = _mm_in(fbonds, W_i)
    for _ in range(2):
        gs_p = _gather_sum(msg, bT)
        msg = _mm_h(fbonds, gs_p, W_i, W_h)
    ga_p = _gather_sum(msg, aT)
    return _mm_out(fatoms, ga_p, W_o[:, :AF], W_o[:, AF:], b_o.reshape(1, H))


# speedup vs baseline: 7.6226x; 1.0964x over previous
import functools

import jax
import jax.numpy as jnp
from jax import lax
from jax.experimental import pallas as pl
from jax.experimental.pallas import tpu as pltpu
from jax.experimental.pallas import tpu_sc as plsc

_NC = 2
_NS = 16
_NW = _NC * _NS
_LANES = 16
_CHUNK = 128


def _mm_in(fbonds, W_i, block_rows=4000, interpret=False):
    E, K = fbonds.shape
    H = W_i.shape[0]
    grid = E // block_rows

    def body(x_ref, w_ref, m_ref):
        b = lax.dot_general(
            x_ref[...], w_ref[...], (((1,), (1,)), ((), ())),
            preferred_element_type=jnp.float32)
        m_ref[...] = jnp.maximum(b, 0.0)

    return pl.pallas_call(
        body,
        grid=(grid,),
        in_specs=[
            pl.BlockSpec((block_rows, K), lambda i: (i, 0)),
            pl.BlockSpec((H, K), lambda i: (0, 0)),
        ],
        out_specs=pl.BlockSpec((block_rows, H), lambda i: (i, 0)),
        out_shape=jax.ShapeDtypeStruct((E, H), jnp.float32),
        interpret=interpret,
    )(fbonds, W_i)


def _mm_h(fbonds, gs_p, W_i, W_h, block_rows=4000, interpret=False):
    E, K = fbonds.shape
    H = W_h.shape[0]
    grid = E // block_rows

    def body(x_ref, g_ref, wi_ref, wh_ref, o_ref):
        b = lax.dot_general(
            x_ref[...], wi_ref[...], (((1,), (1,)), ((), ())),
            preferred_element_type=jnp.float32)
        o_ref[...] = jnp.maximum(
            b + lax.dot_general(
                g_ref[...], wh_ref[...], (((1,), (1,)), ((), ())),
                preferred_element_type=jnp.float32),
            0.0)

    return pl.pallas_call(
        body,
        grid=(grid,),
        in_specs=[
            pl.BlockSpec((block_rows, K), lambda i: (i, 0)),
            pl.BlockSpec((block_rows, H), lambda i: (i, 0)),
            pl.BlockSpec((H, K), lambda i: (0, 0)),
            pl.BlockSpec((H, H), lambda i: (0, 0)),
        ],
        out_specs=pl.BlockSpec((block_rows, H), lambda i: (i, 0)),
        out_shape=jax.ShapeDtypeStruct((E, H), jnp.float32),
        interpret=interpret,
    )(fbonds, gs_p, W_i, W_h)


def _mm_out(fatoms, ga_p, WoA, WoH, b_o2, block_rows=4000, interpret=False):
    N, AF = fatoms.shape
    H = WoH.shape[0]
    grid = N // block_rows

    def body(fa_ref, g_ref, wa_ref, wh_ref, b_ref, o_ref):
        i = pl.program_id(0)
        h = lax.dot_general(fa_ref[...], wa_ref[...], (((1,), (1,)), ((), ())),
                            preferred_element_type=jnp.float32)
        h = h + lax.dot_general(g_ref[...], wh_ref[...], (((1,), (1,)), ((), ())),
                                preferred_element_type=jnp.float32)
        h = jnp.maximum(h + b_ref[...], 0.0)
        part = jnp.sum(h, axis=0, keepdims=True)

        @pl.when(i == 0)
        def _zero():
            o_ref[...] = jnp.zeros_like(o_ref)

        o_ref[...] += part

        @pl.when(i == grid - 1)
        def _scale():
            o_ref[...] = o_ref[...] * (1.0 / N)

    return pl.pallas_call(
        body,
        grid=(grid,),
        in_specs=[
            pl.BlockSpec((block_rows, AF), lambda i: (i, 0)),
            pl.BlockSpec((block_rows, H), lambda i: (i, 0)),
            pl.BlockSpec(WoA.shape, lambda i: (0, 0)),
            pl.BlockSpec(WoH.shape, lambda i: (0, 0)),
            pl.BlockSpec((1, H), lambda i: (0, 0)),
        ],
        out_specs=pl.BlockSpec((1, H), lambda i: (0, 0)),
        out_shape=jax.ShapeDtypeStruct((1, H), jnp.float32),
        interpret=interpret,
    )(fatoms, ga_p, WoA, WoH, b_o2)


def _gather_sum(msg, nbrT_p):
    E, H = msg.shape
    NB, Rp = nbrT_p.shape
    C = _CHUNK
    rpw = Rp // _NW
    n_chunks = rpw // C

    mesh = plsc.VectorSubcoreMesh(core_axis_name="c", subcore_axis_name="s")

    @functools.partial(
        pl.kernel,
        out_type=jax.ShapeDtypeStruct((Rp, H), jnp.float32),
        mesh=mesh,
        scratch_types=[
            pltpu.VMEM((2, NB, C), jnp.int32),
            pltpu.VMEM((2, C, H), jnp.float32),
            [pltpu.SemaphoreType.DMA] * 8,
        ],
    )
    def k(msg_hbm, nbr_hbm, out_hbm, idxt_v, buf_v, sems):
        isem = sems[0:2]
        g0sem = sems[2:4]
        gsem = sems[4:6]
        wsem = sems[6:8]
        wid = lax.axis_index("s") * _NC + lax.axis_index("c")
        base = wid * rpw
        n = n_chunks

        def r0_of(t):
            return pl.multiple_of(base + C * jnp.minimum(t, n - 1), C)

        def wait_idx(sem, s):
            pltpu.make_async_copy(
                nbr_hbm.at[:, pl.ds(base, C)], idxt_v.at[s], sem).wait()

        def wait_gather(sem, s):
            pltpu.make_async_copy(
                out_hbm.at[pl.ds(base, C), :], buf_v.at[s], sem).wait()

        def wait_wb(sem, s):
            pltpu.make_async_copy(
                buf_v.at[s], out_hbm.at[pl.ds(base, C), :], sem).wait()

        def fire_g0(t, s):
            return pltpu.async_copy(
                msg_hbm.at[idxt_v.at[s, 0]], buf_v.at[s], g0sem[s])

        def fire_adds(s):
            for kk in range(1, NB):
                pltpu.async_copy(msg_hbm.at[idxt_v.at[s, kk]], buf_v.at[s],
                                 gsem[s], add=True)

        def half(tt, s, guard):
            o = 1 - s
            wait_idx(isem[o], o)
            if guard is None:
                wait_wb(wsem[o], o)
            else:
                @pl.when(guard)
                def _():
                    wait_wb(wsem[o], o)
            fire_g0(tt + 1, o)
            for _ in range(NB - 1):
                wait_gather(gsem[s], s)
            pltpu.async_copy(buf_v.at[s], out_hbm.at[pl.ds(r0_of(tt), C), :],
                             wsem[s])
            pltpu.async_copy(nbr_hbm.at[:, pl.ds(r0_of(tt + 2), C)],
                             idxt_v.at[s], isem[s])
            wait_gather(g0sem[o], o)
            fire_adds(o)

        pltpu.sync_copy(nbr_hbm.at[:, pl.ds(base, C)], idxt_v.at[0])
        fire_g0(0, 0).wait()
        fire_adds(0)
        pltpu.async_copy(nbr_hbm.at[:, pl.ds(r0_of(1), C)], idxt_v.at[1],
                         isem[1])

        def pair_body(t2, carry):
            half(2 * t2, 0, guard=t2 > 0)
            half(2 * t2 + 1, 1, guard=None)
            return carry

        lax.fori_loop(0, n // 2, pair_body, 0)
        if n % 2:
            half(n - 1, (n - 1) % 2, guard=None)

        s_last = (n - 1) % 2
        o_last = 1 - s_last
        for _ in range(NB - 1):
            wait_gather(gsem[o_last], o_last)
        wait_wb(wsem[s_last], s_last)
        wait_idx(isem[s_last], s_last)

    return k(msg, nbrT_p)


def kernel(fatoms, fbonds, agraph, bgraph, W_i, W_h, W_o, b_o):
    N, AF = fatoms.shape
    E, K = fbonds.shape
    H = W_i.shape[0]
    NB = bgraph.shape[1]
    tile = _NW * _CHUNK
    Ep = -(-E // tile) * tile
    Np = -(-N // tile) * tile

    padB = jnp.arange(NB * (Ep - E), dtype=jnp.int32).reshape(NB, Ep - E) % E
    padA = jnp.arange(NB * (Np - N), dtype=jnp.int32).reshape(NB, Np - N) % E
    bT = jnp.concatenate([bgraph.astype(jnp.int32).T, padB], axis=1)
    aT = jnp.concatenate([agraph.astype(jnp.int32).T, padA], axis=1)

    msg = _mm_in(fbonds, W_i)
    for _ in range(2):
        gs_p = _gather_sum(msg, bT)
        msg = _mm_h(fbonds, gs_p, W_i, W_h)
    ga_p = _gather_sum(msg, aT)
    return _mm_out(fatoms, ga_p, W_o[:, :AF], W_o[:, AF:], b_o.reshape(1, H))
